# Initial kernel scaffold; baseline (speedup 1.0000x reference)
import jax
import jax.numpy as jnp
from jax.experimental import pallas as pl


def kernel(x, table, W, b):
    raise NotImplementedError("write your pallas kernel here")


# speedup vs baseline: 83.7786x; 83.7786x over previous
import dataclasses
import functools

import jax
import jax.numpy as jnp
from jax import lax
from jax.experimental import pallas as pl
from jax.experimental.pallas import tpu as pltpu
from jax.experimental.pallas import tpu_sc as plsc

_VOCAB = 100000
_EMB = 16
_BATCH = 16384
_HIST = 200

_NC = 2
_NS = 16
_LANES = 16
_NW = _NC * _NS
_BPW = _BATCH // _NW
_CB = 16
_NCHUNK = _BPW // _CB
_CHUNK_TOK = _CB * _HIST

_MAGIC = 12582912.0


def _tw_tc_kernel(table_ref, m_ref, o_ref):
    o_ref[...] = jnp.dot(table_ref[...], m_ref[...],
                         preferred_element_type=jnp.float32)


def _sc_body(tw_hbm, x_hbm, b_hbm, out_hbm,
             tw_v, xbuf0, xbuf1, b_v, out_v, sem0, sem1):
    cid = lax.axis_index("c")
    sid = lax.axis_index("s")
    wid = cid * _NS + sid

    pltpu.sync_copy(tw_hbm, tw_v)
    pltpu.sync_copy(b_hbm, b_v)

    tok0 = wid * (_BPW * _HIST)
    lane = jnp.arange(_LANES, dtype=jnp.int32)
    rowbase = lane * _HIST
    bvec = b_v[...]

    def start(chunk, buf_ref, sem):
        pltpu.async_copy(
            x_hbm.at[pl.ds(tok0 + chunk * _CHUNK_TOK, _CHUNK_TOK)],
            buf_ref, sem)

    def compute(chunk, buf_ref):
        def jstep(j, acc):
            col = plsc.load_gather(buf_ref, [rowbase + j])
            return acc + plsc.load_gather(tw_v, [col])
        acc = lax.fori_loop(0, _HIST, jstep,
                            jnp.zeros((_LANES,), jnp.float32))
        z = acc + bvec
        p = 1.0 / (1.0 + jnp.exp(-z))
        t = p * 10000.0
        r = (t + _MAGIC) - _MAGIC
        out_v[pl.ds(chunk * _CB, _CB)] = r * 0.0001

    start(0, xbuf0, sem0)

    @pl.loop(0, _NCHUNK, step=2)
    def _(c):
        pltpu.make_async_copy(
            x_hbm.at[pl.ds(tok0 + c * _CHUNK_TOK, _CHUNK_TOK)],
            xbuf0, sem0).wait()
        start(c + 1, xbuf1, sem1)
        compute(c, xbuf0)
        pltpu.make_async_copy(
            x_hbm.at[pl.ds(tok0 + (c + 1) * _CHUNK_TOK, _CHUNK_TOK)],
            xbuf1, sem1).wait()

        @pl.when(c + 2 < _NCHUNK)
        def _():
            start(c + 2, xbuf0, sem0)

        compute(c + 1, xbuf1)

    pltpu.sync_copy(out_v, out_hbm.at[pl.ds(wid * _BPW, _BPW)])


@jax.jit
def kernel(x, table, W, b):
    table_r = table.reshape(_VOCAB // 8, 128)
    m = jnp.kron(jnp.eye(8, dtype=jnp.float32),
                 W.astype(jnp.float32) * (1.0 / _HIST))
    tw = pl.pallas_call(
        _tw_tc_kernel,
        out_shape=jax.ShapeDtypeStruct((_VOCAB // 8, 8), jnp.float32),
    )(table_r, m)
    tw = tw.reshape(_VOCAB)

    x_flat = x.reshape(_BATCH * _HIST).astype(jnp.int32)
    b16 = jnp.broadcast_to(b.astype(jnp.float32), (_LANES,))

    mesh = plsc.VectorSubcoreMesh(core_axis_name="c", subcore_axis_name="s",
                                  num_cores=_NC, num_subcores=_NS)
    cp = pltpu.CompilerParams()
    if "needs_layout_passes" in pltpu.CompilerParams.__dataclass_fields__:
        cp = dataclasses.replace(cp, needs_layout_passes=False)
    sc = pl.kernel(
        _sc_body,
        out_type=jax.ShapeDtypeStruct((_BATCH,), jnp.float32),
        mesh=mesh,
        scratch_types=[
            pltpu.VMEM((_VOCAB,), jnp.float32),
            pltpu.VMEM((_CHUNK_TOK,), jnp.int32),
            pltpu.VMEM((_CHUNK_TOK,), jnp.int32),
            pltpu.VMEM((_LANES,), jnp.float32),
            pltpu.VMEM((_BPW,), jnp.float32),
            pltpu.SemaphoreType.DMA,
            pltpu.SemaphoreType.DMA,
        ],
        compiler_params=cp,
    )
    probs = sc(tw, x_flat, b16)
    return probs.reshape(_BATCH, 1)


# speedup vs baseline: 91.3251x; 1.0901x over previous
import dataclasses
import functools

import jax
import jax.numpy as jnp
from jax import lax
from jax.experimental import pallas as pl
from jax.experimental.pallas import tpu as pltpu
from jax.experimental.pallas import tpu_sc as plsc

_VOCAB = 100000
_EMB = 16
_BATCH = 16384
_HIST = 200

_NC = 2
_NS = 16
_LANES = 16
_NW = _NC * _NS
_BPW = _BATCH // _NW
_CB = 64
_NCHUNK = _BPW // _CB
_NGRP = _CB // _LANES
_UNROLL = 8

_TWR = _VOCAB // 8

_MAGIC = 12582912.0


def _tw_tc_kernel(table_ref, m_ref, o_ref):
    o_ref[0] = jnp.dot(table_ref[0], m_ref[...],
                       preferred_element_type=jnp.float32)


def _sc_body(tw_hbm, x_hbm, b_hbm, out_hbm,
             tw_v, xbuf0, xbuf1, b_v, out_v, sem_tw, sem0, sem1):
    cid = lax.axis_index("c")
    sid = lax.axis_index("s")
    wid = cid * _NS + sid
    row0 = wid * _BPW
    tok0 = row0 * _HIST
    ctok = _CB * _HIST

    tw_copy = pltpu.async_copy(tw_hbm, tw_v, sem_tw)
    pltpu.sync_copy(b_hbm, b_v)

    xbufs = (xbuf0, xbuf1)
    sems = (sem0, sem1)

    def start(chunk):
        pltpu.async_copy(
            x_hbm.at[pl.ds(tok0 + chunk * ctok, ctok)],
            xbufs[chunk % 2], sems[chunk % 2])

    def wait(chunk):
        pltpu.make_async_copy(
            x_hbm.at[pl.ds(tok0 + chunk * ctok, ctok)],
            xbufs[chunk % 2], sems[chunk % 2]).wait()

    start(0)
    start(1)
    tw_copy.wait()

    lane = jnp.arange(_LANES, dtype=jnp.int32)
    bvec = b_v[...]

    for c in range(_NCHUNK):
        wait(c)
        xb = xbufs[c % 2]
        for g in range(_NGRP):
            rowbase = lane * _HIST + (g * _LANES * _HIST)

            def jstep(jj, accs, xb=xb, rowbase=rowbase):
                a0, a1 = accs
                for u in range(_UNROLL):
                    j = jj * _UNROLL + u
                    col = plsc.load_gather(xb, [rowbase + j])
                    val = plsc.load_gather(tw_v, [col])
                    if u % 2 == 0:
                        a0 = a0 + val
                    else:
                        a1 = a1 + val
                return a0, a1

            zero = jnp.zeros((_LANES,), jnp.float32)
            a0, a1 = lax.fori_loop(0, _HIST // _UNROLL, jstep, (zero, zero))
            z = (a0 + a1) + bvec
            p = 1.0 / (1.0 + jnp.exp(-z))
            t = p * 10000.0
            r = (t + _MAGIC) - _MAGIC
            out_v[pl.ds(c * _CB + g * _LANES, _LANES)] = r * 0.0001
        if c + 2 < _NCHUNK:
            start(c + 2)

    pltpu.sync_copy(out_v, out_hbm.at[pl.ds(row0, _BPW)])


@jax.jit
def kernel(x, table, W, b):
    grid = 10
    table_r = table.reshape(grid, _TWR // grid, 128)
    m = jnp.kron(jnp.eye(8, dtype=jnp.float32),
                 W.astype(jnp.float32) * (1.0 / _HIST))
    tw = pl.pallas_call(
        _tw_tc_kernel,
        grid=(grid,),
        in_specs=[
            pl.BlockSpec((1, _TWR // grid, 128), lambda i: (i, 0, 0)),
            pl.BlockSpec((128, 8), lambda i: (0, 0)),
        ],
        out_specs=pl.BlockSpec((1, _TWR // grid, 8), lambda i: (i, 0, 0)),
        out_shape=jax.ShapeDtypeStruct((grid, _TWR // grid, 8), jnp.float32),
    )(table_r, m)
    tw = tw.reshape(_VOCAB)

    b16 = jnp.broadcast_to(b.astype(jnp.float32), (_LANES,))

    mesh = plsc.VectorSubcoreMesh(core_axis_name="c", subcore_axis_name="s",
                                  num_cores=_NC, num_subcores=_NS)
    cp = pltpu.CompilerParams()
    if "needs_layout_passes" in pltpu.CompilerParams.__dataclass_fields__:
        cp = dataclasses.replace(cp, needs_layout_passes=False)
    sc = pl.kernel(
        _sc_body,
        out_type=jax.ShapeDtypeStruct((_BATCH,), jnp.float32),
        mesh=mesh,
        scratch_types=[
            pltpu.VMEM((_VOCAB,), jnp.float32),
            pltpu.VMEM((_CB * _HIST,), jnp.int32),
            pltpu.VMEM((_CB * _HIST,), jnp.int32),
            pltpu.VMEM((_LANES,), jnp.float32),
            pltpu.VMEM((_BPW,), jnp.float32),
            pltpu.SemaphoreType.DMA,
            pltpu.SemaphoreType.DMA,
            pltpu.SemaphoreType.DMA,
        ],
        compiler_params=cp,
    )
    x_flat = x.reshape(_BATCH * _HIST).astype(jnp.int32)
    probs = sc(tw, x_flat, b16)
    return probs.reshape(_BATCH, 1)


# speedup vs baseline: 301.0300x; 3.2962x over previous
import dataclasses
import functools

import jax
import jax.numpy as jnp
from jax import lax
from jax.experimental import pallas as pl
from jax.experimental.pallas import tpu as pltpu
from jax.experimental.pallas import tpu_sc as plsc

_VOCAB = 100000
_EMB = 16
_BATCH = 16384
_HIST = 200

_NC = 2
_NS = 16
_LANES = 16
_NW = _NC * _NS
_BPW = _BATCH // _NW
_CCOL = 128
_NCHUNK = _BPW // _CCOL
_NGRP = _CCOL // _LANES

_MAGIC = 12582912.0


def _tw_tc_kernel(tab_ref, w_ref, o_ref):
    part = jnp.sum(tab_ref[...] * w_ref[...], axis=0)

    @pl.when(pl.program_id(0) == 0)
    def _():
        o_ref[...] = part

    @pl.when(pl.program_id(0) != 0)
    def _():
        o_ref[...] += part


def _sc_body(tw_hbm, xt_hbm, b_hbm, out_hbm,
             tw_v, xbuf0, xbuf1, b_v, out_v, sem_tw, sem0, sem1):
    cid = lax.axis_index("c")
    sid = lax.axis_index("s")
    wid = cid * _NS + sid
    col0 = wid * _BPW

    tw_copy = pltpu.async_copy(tw_hbm, tw_v, sem_tw)
    pltpu.sync_copy(b_hbm, b_v)

    xbufs = (xbuf0, xbuf1)
    sems = (sem0, sem1)
    _R0, _R1 = 96, 104
    nsub = 2 * _NCHUNK

    def src(k):
        c, h = k // 2, k % 2
        return xt_hbm.at[pl.ds(h * _R0, _R1 if h else _R0),
                         pl.ds(col0 + c * _CCOL, _CCOL)]

    def dst(k):
        rows = _R1 if k % 2 else _R0
        return xbufs[k % 2].at[pl.ds(0, rows), :]

    def start(k):
        pltpu.async_copy(src(k), dst(k), sems[k % 2])

    def wait(k):
        pltpu.make_async_copy(src(k), dst(k), sems[k % 2]).wait()

    start(0)
    start(1)
    tw_copy.wait()

    bvec = b_v[...]
    zero = jnp.zeros((_LANES,), jnp.float32)
    accs = (zero,) * _NGRP

    for k in range(nsub):
        c, h = k // 2, k % 2
        wait(k)
        xb = xbufs[k % 2]
        rows = _R1 if h else _R0

        def jstep(j, a, xb=xb):
            return tuple(
                a[g] + plsc.load_gather(
                    tw_v, [xb[j, pl.ds(g * _LANES, _LANES)]])
                for g in range(_NGRP))

        accs = lax.fori_loop(0, rows, jstep, accs)
        if k + 2 < nsub:
            start(k + 2)
        if h == 1:
            for g in range(_NGRP):
                z = accs[g] + bvec
                p = 1.0 / (1.0 + jnp.exp(-z))
                t = p * 10000.0
                r = (t + _MAGIC) - _MAGIC
                out_v[pl.ds(c * _CCOL + g * _LANES, _LANES)] = r * 0.0001
            accs = (zero,) * _NGRP

    pltpu.sync_copy(out_v, out_hbm.at[pl.ds(col0, _BPW)])


@jax.jit
def kernel(x, table, W, b):
    tab_t = jnp.swapaxes(table, 0, 1)
    x_t = jnp.swapaxes(x, 0, 1).astype(jnp.int32)
    w_scaled = W.astype(jnp.float32) * (1.0 / _HIST)

    tw = pl.pallas_call(
        _tw_tc_kernel,
        grid=(2,),
        in_specs=[
            pl.BlockSpec((8, _VOCAB), lambda i: (i, 0)),
            pl.BlockSpec((8, 1), lambda i: (i, 0)),
        ],
        out_specs=pl.BlockSpec((_VOCAB,), lambda i: (0,)),
        out_shape=jax.ShapeDtypeStruct((_VOCAB,), jnp.float32),
    )(tab_t, w_scaled)

    b16 = jnp.broadcast_to(b.astype(jnp.float32), (_LANES,))

    mesh = plsc.VectorSubcoreMesh(core_axis_name="c", subcore_axis_name="s",
                                  num_cores=_NC, num_subcores=_NS)
    cp = pltpu.CompilerParams()
    if "needs_layout_passes" in pltpu.CompilerParams.__dataclass_fields__:
        cp = dataclasses.replace(cp, needs_layout_passes=False)
    sc = pl.kernel(
        _sc_body,
        out_type=jax.ShapeDtypeStruct((_BATCH,), jnp.float32),
        mesh=mesh,
        scratch_types=[
            pltpu.VMEM((_VOCAB,), jnp.float32),
            pltpu.VMEM((104, _CCOL), jnp.int32),
            pltpu.VMEM((104, _CCOL), jnp.int32),
            pltpu.VMEM((_LANES,), jnp.float32),
            pltpu.VMEM((_BPW,), jnp.float32),
            pltpu.SemaphoreType.DMA,
            pltpu.SemaphoreType.DMA,
            pltpu.SemaphoreType.DMA,
        ],
        compiler_params=cp,
    )
    probs = sc(tw, x_t, b16)
    return probs.reshape(_BATCH, 1)
